# baseline (device time: 542524 ns/iter reference)
import functools

import jax
import jax.numpy as jnp
from jax import lax
from jax.experimental import pallas as pl
from jax.experimental.pallas import tpu as pltpu

N_X = 2


def kernel(x):
    m, n = x.shape
    n_half = n // N_X

    def body(x_ref, out_ref, stage_ref, stage_sem, local_sem, send_sem, recv_sem):
        my_x = lax.axis_index("x")
        my_y = lax.axis_index("y")
        my_z = lax.axis_index("z")
        other = 1 - my_x
        nbr = (other, my_y, my_z)

        barrier_sem = pltpu.get_barrier_semaphore()
        pl.semaphore_signal(
            barrier_sem, inc=1, device_id=nbr,
            device_id_type=pl.DeviceIdType.MESH,
        )
        pl.semaphore_wait(barrier_sem, 1)

        stage = pltpu.make_async_copy(
            x_ref.at[:, pl.ds(other * n_half, n_half)],
            stage_ref,
            stage_sem,
        )
        stage.start()
        stage.wait()

        rdma = pltpu.make_async_remote_copy(
            src_ref=stage_ref,
            dst_ref=out_ref.at[pl.ds(my_x * m, m), :],
            send_sem=send_sem,
            recv_sem=recv_sem,
            device_id=nbr,
            device_id_type=pl.DeviceIdType.MESH,
        )
        rdma.start()

        local = pltpu.make_async_copy(
            x_ref.at[:, pl.ds(my_x * n_half, n_half)],
            out_ref.at[pl.ds(my_x * m, m), :],
            local_sem,
        )
        local.start()
        local.wait()

        rdma.wait()

        @functools.partial(pl.run_scoped, sem=pltpu.SemaphoreType.REGULAR)
        def _(sem):
            pl.semaphore_signal(
                sem, inc=1, device_id=nbr,
                device_id_type=pl.DeviceIdType.MESH,
            )
            pl.semaphore_wait(sem, 1)

    return pl.pallas_call(
        body,
        out_shape=jax.ShapeDtypeStruct((N_X * m, n_half), x.dtype),
        in_specs=[pl.BlockSpec(memory_space=pl.ANY)],
        out_specs=pl.BlockSpec(memory_space=pl.ANY),
        scratch_shapes=[
            pltpu.VMEM((m, n_half), x.dtype),
            pltpu.SemaphoreType.DMA,
            pltpu.SemaphoreType.DMA,
            pltpu.SemaphoreType.DMA,
            pltpu.SemaphoreType.DMA,
        ],
        compiler_params=pltpu.CompilerParams(collective_id=0),
    )(x)


# device time: 209187 ns/iter; 2.5935x vs baseline; 2.5935x over previous
import functools

import jax
import jax.numpy as jnp
from jax import lax
from jax.experimental import pallas as pl
from jax.experimental.pallas import tpu as pltpu

N_X = 2


def kernel(x):
    m, n = x.shape
    n_half = n // N_X

    def body(x_ref, out_ref, vmem_ref, up_sem, down_sem, send_sem, recv_sem):
        my_x = lax.axis_index("x")
        my_y = lax.axis_index("y")
        my_z = lax.axis_index("z")
        other = 1 - my_x
        nbr = (other, my_y, my_z)

        barrier_sem = pltpu.get_barrier_semaphore()
        pl.semaphore_signal(
            barrier_sem, inc=1, device_id=nbr,
            device_id_type=pl.DeviceIdType.MESH,
        )
        pl.semaphore_wait(barrier_sem, 1)

        rdma = pltpu.make_async_remote_copy(
            src_ref=x_ref.at[:, pl.ds(other * n_half, n_half)],
            dst_ref=out_ref.at[pl.ds(my_x * m, m), :],
            send_sem=send_sem,
            recv_sem=recv_sem,
            device_id=nbr,
            device_id_type=pl.DeviceIdType.MESH,
        )
        rdma.start()

        up = pltpu.make_async_copy(
            x_ref.at[:, pl.ds(my_x * n_half, n_half)],
            vmem_ref,
            up_sem,
        )
        up.start()
        up.wait()
        down = pltpu.make_async_copy(
            vmem_ref,
            out_ref.at[pl.ds(my_x * m, m), :],
            down_sem,
        )
        down.start()
        down.wait()

        rdma.wait()

        @functools.partial(pl.run_scoped, sem=pltpu.SemaphoreType.REGULAR)
        def _(sem):
            pl.semaphore_signal(
                sem, inc=1, device_id=nbr,
                device_id_type=pl.DeviceIdType.MESH,
            )
            pl.semaphore_wait(sem, 1)

    return pl.pallas_call(
        body,
        out_shape=jax.ShapeDtypeStruct((N_X * m, n_half), x.dtype),
        in_specs=[pl.BlockSpec(memory_space=pl.ANY)],
        out_specs=pl.BlockSpec(memory_space=pl.ANY),
        scratch_shapes=[
            pltpu.VMEM((m, n_half), x.dtype),
            pltpu.SemaphoreType.DMA,
            pltpu.SemaphoreType.DMA,
            pltpu.SemaphoreType.DMA,
            pltpu.SemaphoreType.DMA,
        ],
        compiler_params=pltpu.CompilerParams(collective_id=0),
    )(x)
